# baseline (speedup 1.0000x reference)
import functools

import jax
import jax.numpy as jnp
from jax import lax
from jax.experimental import pallas as pl
from jax.experimental.pallas import tpu as pltpu
from jax.experimental.pallas import tpu_sc as plsc

CLS = 1_000_000
BATCH = 16384
FEAT = 32
NC = 2
NS = 16
NW = NC * NS
BPW = BATCH // NW
ROWS = BATCH // 128
CROWS = ROWS // NS
PROWS = ROWS // NW
GROUPS = BPW // 16


def _sc_body(label2d, featT, scentT, t_out,
             idx_c, idx_p, val_v, counts_v, featT_v, scentT_v, t_v,
             hist, sem_f, sem_g, sem_s):
    c = lax.axis_index("c")
    s = lax.axis_index("s")
    wid = s * NC + c
    base = wid * BPW

    pltpu.sync_copy(label2d.at[pl.ds(s * CROWS, CROWS)], idx_c)
    pltpu.sync_copy(label2d.at[pl.ds(wid * PROWS, PROWS)], idx_p)

    fcp = pltpu.async_copy(featT.at[:, pl.ds(base, BPW)], featT_v, sem_f)
    gcp = pltpu.async_copy(scentT.at[:, pl.ds(base, BPW)], scentT_v, sem_g)

    for i in range(128 // 16):
        val_v[pl.ds(i * 16, 16)] = jnp.zeros((16,), jnp.float32)
    zcps = [pltpu.async_copy(val_v, hist.at[idx_c.at[j]], sem_s)
            for j in range(CROWS)]
    for cp in zcps:
        cp.wait()
    plsc.subcore_barrier()

    for i in range(128 // 16):
        val_v[pl.ds(i * 16, 16)] = jnp.ones((16,), jnp.float32)
    acps = [pltpu.async_copy(val_v, hist.at[idx_c.at[j]], sem_s, add=True)
            for j in range(CROWS)]
    for cp in acps:
        cp.wait()
    plsc.subcore_barrier()

    ccps = [pltpu.async_copy(hist.at[idx_p.at[j]], counts_v.at[j], sem_s)
            for j in range(PROWS)]
    fcp.wait()
    gcp.wait()

    def group_body(g, _):
        jrow = g // 8
        lbase = (g % 8) * 16
        accs = [jnp.zeros((16,), jnp.float32) for _ in range(4)]
        for f in range(FEAT):
            a = featT_v[f, pl.ds(g * 16, 16)]
            b = scentT_v[f, pl.ds(g * 16, 16)]
            d = a - b
            accs[f % 4] = accs[f % 4] + d * d
        acc = (accs[0] + accs[1]) + (accs[2] + accs[3])
        t_v[jrow, pl.ds(lbase, 16)] = acc
        return _

    lax.fori_loop(0, GROUPS, group_body, 0, unroll=False)

    for cp in ccps:
        cp.wait()
    for j in range(PROWS):
        for i in range(128 // 16):
            sl = pl.ds(i * 16, 16)
            t_v[j, sl] = t_v[j, sl] / counts_v[j, sl]

    pltpu.sync_copy(t_v, t_out.at[pl.ds(wid * PROWS, PROWS)])


@jax.jit
def _sc_part(label2d, featT, scentT):
    mesh = plsc.VectorSubcoreMesh(core_axis_name="c", subcore_axis_name="s")
    return pl.kernel(
        _sc_body,
        out_type=jax.ShapeDtypeStruct((ROWS, 128), jnp.float32),
        mesh=mesh,
        scratch_types=[
            pltpu.VMEM((CROWS, 128), jnp.int32),
            pltpu.VMEM((PROWS, 128), jnp.int32),
            pltpu.VMEM((128,), jnp.float32),
            pltpu.VMEM((PROWS, 128), jnp.float32),
            pltpu.VMEM((FEAT, BPW), jnp.float32),
            pltpu.VMEM((FEAT, BPW), jnp.float32),
            pltpu.VMEM((PROWS, 128), jnp.float32),
            pltpu.VMEM_SHARED((CLS,), jnp.float32),
            pltpu.SemaphoreType.DMA,
            pltpu.SemaphoreType.DMA,
            pltpu.SemaphoreType.DMA,
        ],
        compiler_params=pltpu.CompilerParams(needs_layout_passes=False),
    )(label2d, featT, scentT)


def _tc_loss_body(t_ref, out_ref):
    out_ref[0, 0] = jnp.sum(jnp.sqrt(t_ref[...])) / BATCH


@jax.jit
def _tc_loss(t):
    return pl.pallas_call(
        _tc_loss_body,
        out_shape=jax.ShapeDtypeStruct((1, 1), jnp.float32),
        in_specs=[pl.BlockSpec(memory_space=pltpu.VMEM)],
        out_specs=pl.BlockSpec(memory_space=pltpu.SMEM),
    )(t)


def kernel(feat, label, centers):
    label = label.astype(jnp.int32)
    label2d = label.reshape(ROWS, 128)
    scent = centers.at[label].get(mode="promise_in_bounds")
    t = _sc_part(label2d, feat.T, scent.T)
    return _tc_loss(t)[0, 0]


# speedup vs baseline: 1.0034x; 1.0034x over previous
import functools

import jax
import jax.numpy as jnp
from jax import lax
from jax.experimental import pallas as pl
from jax.experimental.pallas import tpu as pltpu
from jax.experimental.pallas import tpu_sc as plsc

CLS = 1_000_000
BATCH = 16384
FEAT = 32
NC = 2
NS = 16
NW = NC * NS
BPW = BATCH // NW
ROWS = BATCH // 128
CROWS = ROWS // NS
PROWS = ROWS // NW
GROUPS = BPW // 16


def _sc_body(label2d, featT, scentT, t_out,
             idx_c, idx_p, val_v, counts_v, featT_v, scentT_v, t_v,
             hist, sem_f, sem_g, sem_s):
    c = lax.axis_index("c")
    s = lax.axis_index("s")
    wid = s * NC + c
    base = wid * BPW

    pltpu.sync_copy(label2d.at[pl.ds(s * CROWS, CROWS)], idx_c)
    pltpu.sync_copy(label2d.at[pl.ds(wid * PROWS, PROWS)], idx_p)

    fcp = pltpu.async_copy(featT.at[:, pl.ds(base, BPW)], featT_v, sem_f)
    gcp = pltpu.async_copy(scentT.at[:, pl.ds(base, BPW)], scentT_v, sem_g)

    for i in range(128 // 16):
        val_v[pl.ds(i * 16, 16)] = jnp.zeros((16,), jnp.float32)
    zcps = [pltpu.async_copy(val_v, hist.at[idx_c.at[j]], sem_s)
            for j in range(CROWS)]
    for cp in zcps:
        cp.wait()
    plsc.subcore_barrier()

    for i in range(128 // 16):
        val_v[pl.ds(i * 16, 16)] = jnp.ones((16,), jnp.float32)
    acps = [pltpu.async_copy(val_v, hist.at[idx_c.at[j]], sem_s, add=True)
            for j in range(CROWS)]
    for cp in acps:
        cp.wait()
    plsc.subcore_barrier()

    ccps = [pltpu.async_copy(hist.at[idx_p.at[j]], counts_v.at[j], sem_s)
            for j in range(PROWS)]
    fcp.wait()
    gcp.wait()

    def group_body(g, _):
        jrow = g // 8
        lbase = (g % 8) * 16
        acc = jnp.zeros((16,), jnp.float32)
        for f in range(FEAT):
            a = featT_v[f, pl.ds(g * 16, 16)]
            b = scentT_v[f, pl.ds(g * 16, 16)]
            d = a - b
            acc = acc + d * d
        t_v[jrow, pl.ds(lbase, 16)] = acc
        return _

    lax.fori_loop(0, GROUPS, group_body, 0, unroll=False)

    for cp in ccps:
        cp.wait()
    for j in range(PROWS):
        for i in range(128 // 16):
            sl = pl.ds(i * 16, 16)
            t_v[j, sl] = t_v[j, sl] / counts_v[j, sl]

    pltpu.sync_copy(t_v, t_out.at[pl.ds(wid * PROWS, PROWS)])


@jax.jit
def _sc_part(label2d, featT, scentT):
    mesh = plsc.VectorSubcoreMesh(core_axis_name="c", subcore_axis_name="s")
    return pl.kernel(
        _sc_body,
        out_type=jax.ShapeDtypeStruct((ROWS, 128), jnp.float32),
        mesh=mesh,
        scratch_types=[
            pltpu.VMEM((CROWS, 128), jnp.int32),
            pltpu.VMEM((PROWS, 128), jnp.int32),
            pltpu.VMEM((128,), jnp.float32),
            pltpu.VMEM((PROWS, 128), jnp.float32),
            pltpu.VMEM((FEAT, BPW), jnp.float32),
            pltpu.VMEM((FEAT, BPW), jnp.float32),
            pltpu.VMEM((PROWS, 128), jnp.float32),
            pltpu.VMEM_SHARED((CLS,), jnp.float32),
            pltpu.SemaphoreType.DMA,
            pltpu.SemaphoreType.DMA,
            pltpu.SemaphoreType.DMA,
        ],
        compiler_params=pltpu.CompilerParams(needs_layout_passes=False),
    )(label2d, featT, scentT)


def _tc_loss_body(t_ref, out_ref):
    out_ref[0, 0] = jnp.sum(jnp.sqrt(t_ref[...])) / BATCH


@jax.jit
def _tc_loss(t):
    return pl.pallas_call(
        _tc_loss_body,
        out_shape=jax.ShapeDtypeStruct((1, 1), jnp.float32),
        in_specs=[pl.BlockSpec(memory_space=pltpu.VMEM)],
        out_specs=pl.BlockSpec(memory_space=pltpu.SMEM),
    )(t)


def kernel(feat, label, centers):
    label = label.astype(jnp.int32)
    label2d = label.reshape(ROWS, 128)
    scent = centers.at[label].get(mode="promise_in_bounds")
    t = _sc_part(label2d, feat.T, scent.T)
    return _tc_loss(t)[0, 0]
